# baseline (speedup 1.0000x reference)
import functools

import numpy as np
import jax
import jax.numpy as jnp
from jax import lax
from jax.experimental import pallas as pl
from jax.experimental.pallas import tpu as pltpu
from jax.experimental.pallas import tpu_sc as plsc

F32 = jnp.float32
I32 = jnp.int32

B = 4
NQ = 256
NH = 8
NL = 4
NP = 4
HID = 256
DH = 32
SIZES = (128, 64, 32, 16)
CDIMS = (128, 256, 512, 1024)
NW = 32
QPW = NQ // NW


def _fold_body(wv_ref, bias_ref, wval_ref, bval_ref, m_ref, c_ref):
    wval = wval_ref[...]
    m_ref[...] = jnp.dot(wv_ref[...], wval, preferred_element_type=F32)
    c_ref[...] = jnp.dot(bias_ref[...], wval, preferred_element_type=F32) + bval_ref[...]


def _fold_call(wv_cat, bias8, w_value, b_value):
    return pl.pallas_call(
        _fold_body,
        out_shape=(
            jax.ShapeDtypeStruct((sum(CDIMS), HID), F32),
            jax.ShapeDtypeStruct((8, HID), F32),
        ),
    )(wv_cat, bias8, w_value, b_value)


def _val_body(s, x_ref, xt_ref, m_ref, c_ref, o_ref):
    acc = lax.dot_general(x_ref[0], m_ref[...], (((0,), (0,)), ((), ())),
                          preferred_element_type=F32) + c_ref[...]
    acct = lax.dot_general(xt_ref[0, :, :s], m_ref[...], (((0,), (0,)), ((), ())),
                           preferred_element_type=F32) + c_ref[...]
    R = acc.shape[0]
    for h in range(NH):
        a_h = acc[:, h * DH:(h + 1) * DH]
        nxt = jnp.concatenate([a_h[s:], acct[:, h * DH:(h + 1) * DH]], axis=0)
        o_ref[0, h] = jnp.concatenate([a_h, nxt], axis=1)


def _val_call(x, m, c, R, S):
    _, C, HW = x.shape
    k = R // 128
    jmax = HW // 128 - 1
    return pl.pallas_call(
        functools.partial(_val_body, S),
        grid=(B, HW // R),
        in_specs=[
            pl.BlockSpec((1, C, R), lambda b, i: (b, 0, i)),
            pl.BlockSpec((1, C, 128), lambda b, i: (b, 0, jnp.minimum(i * k + k, jmax))),
            pl.BlockSpec((C, HID), lambda b, i: (0, 0)),
            pl.BlockSpec((1, HID), lambda b, i: (0, 0)),
        ],
        out_specs=pl.BlockSpec((1, NH, R, 2 * DH), lambda b, i: (b, 0, i, 0)),
        out_shape=jax.ShapeDtypeStruct((B, NH, HW, 2 * DH), F32),
    )(x, x, m, c)


def _prep_body(q_ref, wq_ref, bq_ref, wox_ref, box_ref, woy_ref, boy_ref,
               wat_ref, bat_ref, g_ref, cent_ref, bound_ref, sw_ref, hwc_ref,
               hcol_ref, i_ref, w_ref):
    b = pl.program_id(0)
    q = jnp.dot(q_ref[0], wq_ref[...], preferred_element_type=F32) + bq_ref[...]
    ox = jnp.dot(q, wox_ref[...], preferred_element_type=F32) + box_ref[...]
    oy = jnp.dot(q, woy_ref[...], preferred_element_type=F32) + boy_ref[...]
    logit = jnp.dot(q, wat_ref[...], preferred_element_type=F32) + bat_ref[...]
    m = jnp.max(logit, axis=1, keepdims=True)
    e = jnp.exp(logit - m)
    denom = jnp.dot(e, g_ref[...], preferred_element_type=F32)
    a = e / denom

    cent = cent_ref[...]
    bnd = bound_ref[...]
    ix = ox + cent
    iy = oy + cent
    x0 = jnp.floor(ix)
    y0 = jnp.floor(iy)
    fx = ix - x0
    fy = iy - y0
    one = jnp.float32(1.0)
    x1 = x0 + one
    y1 = y0 + one

    vx0 = ((x0 >= 0) & (x0 <= bnd)).astype(F32)
    vx1 = ((x1 >= 0) & (x1 <= bnd)).astype(F32)
    vy0 = ((y0 >= 0) & (y0 <= bnd)).astype(F32)
    vy1 = ((y1 >= 0) & (y1 <= bnd)).astype(F32)

    x0c = jnp.clip(x0, 0.0, bnd)
    x1c = jnp.clip(x1, 0.0, bnd)
    y0c = jnp.clip(y0, 0.0, bnd)
    y1c = jnp.clip(y1, 0.0, bnd)
    yb = jnp.clip(y0, 0.0, bnd - one)
    gA = ((one - fy) * vy0 * (y0c == yb).astype(F32)
          + fy * vy1 * (y1c == yb).astype(F32))
    gB = ((one - fy) * vy0 * (y0c == yb + one).astype(F32)
          + fy * vy1 * (y1c == yb + one).astype(F32))

    base = (b * NH + hcol_ref[...]) * hwc_ref[...] + yb.astype(I32) * sw_ref[...]
    idxs = (base + x0c.astype(I32), base + x1c.astype(I32))
    wts = (a * (one - fx) * vx0 * gA,
           a * (one - fx) * vx0 * gB,
           a * fx * vx1 * gA,
           a * fx * vx1 * gB)

    for l in range(NL):
        sl = slice(l * 32, (l + 1) * 32)
        for c in range(2):
            i_ref[c, 0, l] = idxs[c][:, sl]
        for c in range(4):
            w_ref[c, 0, l] = wts[c][:, sl]


def _prep_call(q3, wq, bq, wox, box, woy, boy, wat, bat, g, cent, bound, sw,
               hwc, hcol):
    full = lambda shape: pl.BlockSpec(shape, lambda b: tuple(0 for _ in shape))
    ispec = pl.BlockSpec((2, 1, NL, NQ, 32), lambda b: (0, b, 0, 0, 0))
    wspec = pl.BlockSpec((4, 1, NL, NQ, 32), lambda b: (0, b, 0, 0, 0))
    return pl.pallas_call(
        _prep_body,
        grid=(B,),
        in_specs=[
            pl.BlockSpec((1, NQ, 2560), lambda b: (b, 0, 0)),
            full((2560, HID)), full((1, HID)),
            full((HID, 128)), full((1, 128)),
            full((HID, 128)), full((1, 128)),
            full((HID, 128)), full((1, 128)),
            full((128, 128)),
            full((1, 128)), full((1, 128)), full((1, 128)), full((1, 128)),
            full((1, 128)),
        ],
        out_specs=(ispec, wspec),
        out_shape=(jax.ShapeDtypeStruct((2, B, NL, NQ, 32), I32),
                   jax.ShapeDtypeStruct((4, B, NL, NQ, 32), F32)),
    )(q3, wq, bq, wox, box, woy, boy, wat, bat, g, cent, bound, sw, hwc, hcol)


_GDN = lax.GatherDimensionNumbers(offset_dims=(), collapsed_slice_dims=(0,),
                                  start_index_map=(0,))


def _lane_bcast(vec, j):
    idx = jnp.full((16, 1), j, dtype=I32)
    return lax.gather(vec, idx, _GDN, (1,),
                      mode=lax.GatherScatterMode.PROMISE_IN_BOUNDS)


_NSTEP = NL * B


def _sc_body(t2, t3, t4, t5, ihbm, whbm, out_ref,
             ivs, wvs, g, acc,
             si0, si1, si2, sw0, sw1, sw2,
             sg00, sg01, sg10, sg11):
    tabs = (t2, t3, t4, t5)
    si = (si0, si1, si2)
    sw = (sw0, sw1, sw2)
    sg = ((sg00, sg01), (sg10, sg11))

    wid = lax.axis_index("s") * 2 + lax.axis_index("c")
    q0m32 = wid * (QPW * 32)

    def stage(t):
        ss = t % 3
        b, l = t % 4, t // 4
        pltpu.async_copy(ihbm.at[:, b, l, pl.ds(q0m32, 256)], ivs.at[ss], si[ss])
        pltpu.async_copy(whbm.at[:, b, l, pl.ds(q0m32, 256)], wvs.at[ss], sw[ss])

    def fire(t):
        ss, gb, l = t % 3, t % 2, t // 4
        pltpu.make_async_copy(ihbm.at[:, 0, 0, pl.ds(0, 256)], ivs.at[ss],
                              si[ss]).wait()
        pltpu.make_async_copy(whbm.at[:, 0, 0, pl.ds(0, 256)], wvs.at[ss],
                              sw[ss]).wait()
        for c in range(2):
            for h4 in range(4):
                pltpu.async_copy(tabs[l].at[ivs.at[ss, c, pl.ds(h4 * 64, 64)]],
                                 g.at[gb, c, pl.ds(h4 * 64, 64)], sg[gb][c])

    def compute(t):
        ss, gb, l, b = t % 3, t % 2, t // 4, t % 4
        for c in range(2):
            pltpu.make_async_copy(tabs[0].at[pl.ds(0, 256)], g.at[gb, c],
                                  sg[gb][c]).wait()

        def kb(k, _):
            wvecs = [wvs[ss, c, pl.ds(k * 16, 16)] for c in range(4)]
            for rr in range(4):
                arow = b * 64 + k * 4 + rr
                lo = []
                hi = []
                for c in range(4):
                    cy, pr = c >> 1, c & 1
                    j0 = rr * 4
                    w0 = _lane_bcast(wvecs[c], j0)
                    cl = w0 * g[gb, cy, k * 16 + j0, pr, pl.ds(0, 16)]
                    ch = w0 * g[gb, cy, k * 16 + j0, pr, pl.ds(16, 16)]
                    for p in range(1, 4):
                        j = rr * 4 + p
                        wj = _lane_bcast(wvecs[c], j)
                        cl = cl + wj * g[gb, cy, k * 16 + j, pr, pl.ds(0, 16)]
                        ch = ch + wj * g[gb, cy, k * 16 + j, pr, pl.ds(16, 16)]
                    lo.append(cl)
                    hi.append(ch)
                al = (lo[0] + lo[1]) + (lo[2] + lo[3])
                ah = (hi[0] + hi[1]) + (hi[2] + hi[3])
                if l > 0:
                    al = al + acc[arow, pl.ds(0, 16)]
                    ah = ah + acc[arow, pl.ds(16, 16)]
                acc[arow, pl.ds(0, 16)] = al
                acc[arow, pl.ds(16, 16)] = ah
            return _

        lax.fori_loop(0, 16, kb, None)

    for t in range(_NSTEP + 2):
        if 1 <= t <= _NSTEP:
            fire(t - 1)
        if t < _NSTEP:
            stage(t)
        if t >= 2:
            compute(t - 2)

    for b in range(B):
        pltpu.sync_copy(acc.at[pl.ds(b * 64, 64)],
                        out_ref.at[pl.ds(b * 2048 + wid * 64, 64)])


def _sc_gather(tabs, ihbm, whbm):
    mesh = plsc.VectorSubcoreMesh(core_axis_name="c", subcore_axis_name="s")
    kern = pl.kernel(
        _sc_body,
        out_type=jax.ShapeDtypeStruct((B * NQ * NH, DH), F32),
        mesh=mesh,
        compiler_params=pltpu.CompilerParams(use_tc_tiling_on_sc=False),
        scratch_types=(
            [pltpu.VMEM((3, 2, 256), I32),
             pltpu.VMEM((3, 4, 256), F32),
             pltpu.VMEM((2, 2, 256, 2, DH), F32),
             pltpu.VMEM((B * 64, DH), F32)]
            + [pltpu.SemaphoreType.DMA for _ in range(10)]
        ),
    )
    return kern(*tabs, ihbm, whbm)


def _out_body(x_ref, wao_ref, bao_ref, wout_ref, bout_ref, o_ref):
    t = jnp.dot(x_ref[...], wao_ref[...], preferred_element_type=F32) + bao_ref[...]
    o_ref[...] = jnp.dot(t, wout_ref[...], preferred_element_type=F32) + bout_ref[...]


def _out_call(x, wao, bao, wout, bout):
    return pl.pallas_call(
        _out_body,
        out_shape=jax.ShapeDtypeStruct((B * NQ, 2560), F32),
    )(x, wao, bao, wout, bout)


_COLS = np.arange(128)
_L_OF = _COLS // 32
_H_OF = (_COLS % 32) // 4
_P_OF = _COLS % 4
_PERM_ATTN = _H_OF * 16 + _L_OF * 4 + _P_OF
_PERM_OFF_X = ((_H_OF * 4 + _L_OF) * 4 + _P_OF) * 2
_PERM_OFF_Y = _PERM_OFF_X + 1
_G_NP = (_H_OF[:, None] == _H_OF[None, :]).astype(np.float32)
_S_NP = np.array(SIZES, np.float32)[_L_OF]
_CENT_NP = (_S_NP / 2.0 - 0.5).astype(np.float32)[None]
_BOUND_NP = (_S_NP - 1.0).astype(np.float32)[None]
_SW_NP = _S_NP.astype(np.int32)[None]
_HWC_NP = (_S_NP * _S_NP).astype(np.int32)[None]
_HCOL_NP = _H_OF.astype(np.int32)[None]


def kernel(queries, res2, res3, res4, res5, W_q, b_q, Wv2, bv2, Wv3, bv3,
           Wv4, bv4, Wv5, bv5, level_embed, W_value, b_value, W_off, b_off,
           W_attn, b_attn, W_ao, b_ao, W_out, b_out):
    wv_cat = jnp.concatenate([Wv2, Wv3, Wv4, Wv5], axis=0)
    bias4 = jnp.stack([bv2, bv3, bv4, bv5]) + level_embed
    bias8 = jnp.concatenate([bias4, jnp.zeros((4, HID), F32)], axis=0)
    m_cat, c8 = _fold_call(wv_cat, bias8, W_value, b_value.reshape(1, HID))

    feats = (res2, res3, res4, res5)
    r_blocks = (2048, 1024, 1024, 256)
    tabs = []
    start = 0
    for l in range(NL):
        C, S = CDIMS[l], SIZES[l]
        m_l = lax.slice(m_cat, (start, 0), (start + C, HID))
        c_l = lax.slice(c8, (l, 0), (l + 1, HID))
        x = feats[l].reshape(B, C, S * S)
        v4 = _val_call(x, m_l, c_l, r_blocks[l], S)
        tabs.append(v4.reshape(B * NH * S * S, 2, DH))
        start += C

    wox = W_off[:, _PERM_OFF_X]
    box = b_off[_PERM_OFF_X].reshape(1, 128)
    woy = W_off[:, _PERM_OFF_Y]
    boy = b_off[_PERM_OFF_Y].reshape(1, 128)
    wat = W_attn[:, _PERM_ATTN]
    bat = b_attn[_PERM_ATTN].reshape(1, 128)
    idx_arr, w_arr = _prep_call(
        queries, W_q, b_q.reshape(1, HID), wox, box, woy, boy,
        wat, bat, jnp.asarray(_G_NP), jnp.asarray(_CENT_NP),
        jnp.asarray(_BOUND_NP), jnp.asarray(_SW_NP),
        jnp.asarray(_HWC_NP), jnp.asarray(_HCOL_NP))

    sampled = _sc_gather(tabs, idx_arr.reshape(2, B, NL, NQ * 32),
                         w_arr.reshape(4, B, NL, NQ * 32))

    out = _out_call(sampled.reshape(B * NQ, HID), W_ao, b_ao.reshape(1, HID),
                    W_out, b_out.reshape(1, 2560))
    return out.reshape(B, NQ, 2560)


# speedup vs baseline: 1.4168x; 1.4168x over previous
import functools

import numpy as np
import jax
import jax.numpy as jnp
from jax import lax
from jax.experimental import pallas as pl
from jax.experimental.pallas import tpu as pltpu
from jax.experimental.pallas import tpu_sc as plsc

F32 = jnp.float32
I32 = jnp.int32

B = 4
NQ = 256
NH = 8
NL = 4
NP = 4
HID = 256
DH = 32
SIZES = (128, 64, 32, 16)
CDIMS = (128, 256, 512, 1024)
NW = 32
QPW = NQ // NW


def _fold_body(wv_ref, bias_ref, wval_ref, bval_ref, m_ref, c_ref):
    wval = wval_ref[...]
    m_ref[...] = jnp.dot(wv_ref[...], wval, preferred_element_type=F32)
    c_ref[...] = jnp.dot(bias_ref[...], wval, preferred_element_type=F32) + bval_ref[...]


def _fold_call(wv_cat, bias8, w_value, b_value):
    return pl.pallas_call(
        _fold_body,
        out_shape=(
            jax.ShapeDtypeStruct((sum(CDIMS), HID), F32),
            jax.ShapeDtypeStruct((8, HID), F32),
        ),
    )(wv_cat, bias8, w_value, b_value)


def _val_body(s, x_ref, xt_ref, m_ref, c_ref, o_ref):
    acc = lax.dot_general(x_ref[0], m_ref[...], (((0,), (0,)), ((), ())),
                          preferred_element_type=F32) + c_ref[...]
    acct = lax.dot_general(xt_ref[0, :, :s], m_ref[...], (((0,), (0,)), ((), ())),
                           preferred_element_type=F32) + c_ref[...]
    R = acc.shape[0]
    nxt = jnp.concatenate([acc[s:], acct], axis=0)
    for hp in range(NH // 2):
        c0 = slice(hp * 2 * DH, hp * 2 * DH + DH)
        c1 = slice(hp * 2 * DH + DH, hp * 2 * DH + 2 * DH)
        o_ref[0, hp] = jnp.concatenate(
            [acc[:, c0], nxt[:, c0], acc[:, c1], nxt[:, c1]], axis=1)


def _val_call(x, m, c, R, S):
    _, C, HW = x.shape
    k = R // 128
    jmax = HW // 128 - 1
    return pl.pallas_call(
        functools.partial(_val_body, S),
        grid=(B, HW // R),
        in_specs=[
            pl.BlockSpec((1, C, R), lambda b, i: (b, 0, i)),
            pl.BlockSpec((1, C, 128), lambda b, i: (b, 0, jnp.minimum(i * k + k, jmax))),
            pl.BlockSpec((C, HID), lambda b, i: (0, 0)),
            pl.BlockSpec((1, HID), lambda b, i: (0, 0)),
        ],
        out_specs=pl.BlockSpec((1, NH // 2, R, 4 * DH), lambda b, i: (b, 0, i, 0)),
        out_shape=jax.ShapeDtypeStruct((B, NH // 2, HW, 4 * DH), F32),
    )(x, x, m, c)


def _prep_body(q_ref, wq_ref, bq_ref, wox_ref, box_ref, woy_ref, boy_ref,
               wat_ref, bat_ref, g_ref, cent_ref, bound_ref, sw_ref, hwc_ref,
               hcol_ref, i_ref, w_ref):
    b = pl.program_id(0)
    q = jnp.dot(q_ref[0], wq_ref[...], preferred_element_type=F32) + bq_ref[...]
    ox = jnp.dot(q, wox_ref[...], preferred_element_type=F32) + box_ref[...]
    oy = jnp.dot(q, woy_ref[...], preferred_element_type=F32) + boy_ref[...]
    logit = jnp.dot(q, wat_ref[...], preferred_element_type=F32) + bat_ref[...]
    m = jnp.max(logit, axis=1, keepdims=True)
    e = jnp.exp(logit - m)
    denom = jnp.dot(e, g_ref[...], preferred_element_type=F32)
    a = e / denom

    cent = cent_ref[...]
    bnd = bound_ref[...]
    ix = ox + cent
    iy = oy + cent
    x0 = jnp.floor(ix)
    y0 = jnp.floor(iy)
    fx = ix - x0
    fy = iy - y0
    one = jnp.float32(1.0)
    x1 = x0 + one
    y1 = y0 + one

    vx0 = ((x0 >= 0) & (x0 <= bnd)).astype(F32)
    vx1 = ((x1 >= 0) & (x1 <= bnd)).astype(F32)
    vy0 = ((y0 >= 0) & (y0 <= bnd)).astype(F32)
    vy1 = ((y1 >= 0) & (y1 <= bnd)).astype(F32)

    x0c = jnp.clip(x0, 0.0, bnd)
    x1c = jnp.clip(x1, 0.0, bnd)
    y0c = jnp.clip(y0, 0.0, bnd)
    y1c = jnp.clip(y1, 0.0, bnd)
    yb = jnp.clip(y0, 0.0, bnd - one)
    gA = ((one - fy) * vy0 * (y0c == yb).astype(F32)
          + fy * vy1 * (y1c == yb).astype(F32))
    gB = ((one - fy) * vy0 * (y0c == yb + one).astype(F32)
          + fy * vy1 * (y1c == yb + one).astype(F32))

    hcol = hcol_ref[...]
    hwc = hwc_ref[...]
    sw = sw_ref[...]
    base = ((b * (NH // 2) + jnp.right_shift(hcol, 1)) * (2 * hwc)
            + yb.astype(I32) * (2 * sw) + jnp.bitwise_and(hcol, 1))
    idxs = (base + x0c.astype(I32) * 2, base + x1c.astype(I32) * 2)
    wts = (a * (one - fx) * vx0 * gA,
           a * (one - fx) * vx0 * gB,
           a * fx * vx1 * gA,
           a * fx * vx1 * gB)

    for l in range(NL):
        sl = slice(l * 32, (l + 1) * 32)
        for c in range(2):
            i_ref[c, 0, l] = idxs[c][:, sl]
        for c in range(4):
            w_ref[c, 0, l] = wts[c][:, sl]


def _prep_call(q3, wq, bq, wox, box, woy, boy, wat, bat, g, cent, bound, sw,
               hwc, hcol):
    full = lambda shape: pl.BlockSpec(shape, lambda b: tuple(0 for _ in shape))
    ispec = pl.BlockSpec((2, 1, NL, NQ, 32), lambda b: (0, b, 0, 0, 0))
    wspec = pl.BlockSpec((4, 1, NL, NQ, 32), lambda b: (0, b, 0, 0, 0))
    return pl.pallas_call(
        _prep_body,
        grid=(B,),
        in_specs=[
            pl.BlockSpec((1, NQ, 2560), lambda b: (b, 0, 0)),
            full((2560, HID)), full((1, HID)),
            full((HID, 128)), full((1, 128)),
            full((HID, 128)), full((1, 128)),
            full((HID, 128)), full((1, 128)),
            full((128, 128)),
            full((1, 128)), full((1, 128)), full((1, 128)), full((1, 128)),
            full((1, 128)),
        ],
        out_specs=(ispec, wspec),
        out_shape=(jax.ShapeDtypeStruct((2, B, NL, NQ, 32), I32),
                   jax.ShapeDtypeStruct((4, B, NL, NQ, 32), F32)),
    )(q3, wq, bq, wox, box, woy, boy, wat, bat, g, cent, bound, sw, hwc, hcol)


_GDN = lax.GatherDimensionNumbers(offset_dims=(), collapsed_slice_dims=(0,),
                                  start_index_map=(0,))


def _lane_bcast(vec, j):
    idx = jnp.full((16, 1), j, dtype=I32)
    return lax.gather(vec, idx, _GDN, (1,),
                      mode=lax.GatherScatterMode.PROMISE_IN_BOUNDS)


_NSTEP = NL * B


def _sc_body(t2, t3, t4, t5, ihbm, whbm, out_ref,
             ivs, wvs, g, acc,
             si0, si1, si2, sw0, sw1, sw2,
             sg00, sg01, sg10, sg11):
    tabs = (t2, t3, t4, t5)
    si = (si0, si1, si2)
    sw = (sw0, sw1, sw2)
    sg = ((sg00, sg01), (sg10, sg11))

    wid = lax.axis_index("s") * 2 + lax.axis_index("c")
    q0m32 = wid * (QPW * 32)

    def stage(t):
        ss = t % 3
        b, l = t % 4, t // 4
        pltpu.async_copy(ihbm.at[:, b, l, pl.ds(q0m32, 256)], ivs.at[ss], si[ss])
        pltpu.async_copy(whbm.at[:, b, l, pl.ds(q0m32, 256)], wvs.at[ss], sw[ss])

    def fire(t):
        ss, gb, l = t % 3, t % 2, t // 4
        pltpu.make_async_copy(ihbm.at[:, 0, 0, pl.ds(0, 256)], ivs.at[ss],
                              si[ss]).wait()
        pltpu.make_async_copy(whbm.at[:, 0, 0, pl.ds(0, 256)], wvs.at[ss],
                              sw[ss]).wait()
        for c in range(2):
            for h4 in range(4):
                pltpu.async_copy(tabs[l].at[ivs.at[ss, c, pl.ds(h4 * 64, 64)]],
                                 g.at[gb, c, pl.ds(h4 * 64, 64)], sg[gb][c])

    def compute(t):
        ss, gb, l, b = t % 3, t % 2, t // 4, t % 4
        for c in range(2):
            pltpu.make_async_copy(tabs[0].at[pl.ds(0, 256)], g.at[gb, c],
                                  sg[gb][c]).wait()

        def kb(k, _):
            wvecs = [wvs[ss, c, pl.ds(k * 16, 16)] for c in range(4)]
            for rr in range(4):
                arow = b * 64 + k * 4 + rr
                lo = []
                hi = []
                for c in range(4):
                    cy, pr = c >> 1, c & 1
                    j0 = rr * 4
                    w0 = _lane_bcast(wvecs[c], j0)
                    cl = w0 * g[gb, cy, k * 16 + j0, pr, pl.ds(0, 16)]
                    ch = w0 * g[gb, cy, k * 16 + j0, pr, pl.ds(16, 16)]
                    for p in range(1, 4):
                        j = rr * 4 + p
                        wj = _lane_bcast(wvecs[c], j)
                        cl = cl + wj * g[gb, cy, k * 16 + j, pr, pl.ds(0, 16)]
                        ch = ch + wj * g[gb, cy, k * 16 + j, pr, pl.ds(16, 16)]
                    lo.append(cl)
                    hi.append(ch)
                al = (lo[0] + lo[1]) + (lo[2] + lo[3])
                ah = (hi[0] + hi[1]) + (hi[2] + hi[3])
                if l > 0:
                    al = al + acc[arow, pl.ds(0, 16)]
                    ah = ah + acc[arow, pl.ds(16, 16)]
                acc[arow, pl.ds(0, 16)] = al
                acc[arow, pl.ds(16, 16)] = ah
            return _

        lax.fori_loop(0, 16, kb, None)

    for t in range(_NSTEP + 2):
        if 1 <= t <= _NSTEP:
            fire(t - 1)
        if t < _NSTEP:
            stage(t)
        if t >= 2:
            compute(t - 2)

    for b in range(B):
        pltpu.sync_copy(acc.at[pl.ds(b * 64, 64)],
                        out_ref.at[pl.ds(b * 2048 + wid * 64, 64)])


def _sc_gather(tabs, ihbm, whbm):
    mesh = plsc.VectorSubcoreMesh(core_axis_name="c", subcore_axis_name="s")
    kern = pl.kernel(
        _sc_body,
        out_type=jax.ShapeDtypeStruct((B * NQ * NH, DH), F32),
        mesh=mesh,
        compiler_params=pltpu.CompilerParams(use_tc_tiling_on_sc=False),
        scratch_types=(
            [pltpu.VMEM((3, 2, 256), I32),
             pltpu.VMEM((3, 4, 256), F32),
             pltpu.VMEM((2, 2, 256, 2, DH), F32),
             pltpu.VMEM((B * 64, DH), F32)]
            + [pltpu.SemaphoreType.DMA for _ in range(10)]
        ),
    )
    return kern(*tabs, ihbm, whbm)


def _out_body(x_ref, wao_ref, bao_ref, wout_ref, bout_ref, o_ref):
    t = jnp.dot(x_ref[...], wao_ref[...], preferred_element_type=F32) + bao_ref[...]
    o_ref[...] = jnp.dot(t, wout_ref[...], preferred_element_type=F32) + bout_ref[...]


def _out_call(x, wao, bao, wout, bout):
    return pl.pallas_call(
        _out_body,
        out_shape=jax.ShapeDtypeStruct((B * NQ, 2560), F32),
    )(x, wao, bao, wout, bout)


_COLS = np.arange(128)
_L_OF = _COLS // 32
_H_OF = (_COLS % 32) // 4
_P_OF = _COLS % 4
_PERM_ATTN = _H_OF * 16 + _L_OF * 4 + _P_OF
_PERM_OFF_X = ((_H_OF * 4 + _L_OF) * 4 + _P_OF) * 2
_PERM_OFF_Y = _PERM_OFF_X + 1
_G_NP = (_H_OF[:, None] == _H_OF[None, :]).astype(np.float32)
_S_NP = np.array(SIZES, np.float32)[_L_OF]
_CENT_NP = (_S_NP / 2.0 - 0.5).astype(np.float32)[None]
_BOUND_NP = (_S_NP - 1.0).astype(np.float32)[None]
_SW_NP = _S_NP.astype(np.int32)[None]
_HWC_NP = (_S_NP * _S_NP).astype(np.int32)[None]
_HCOL_NP = _H_OF.astype(np.int32)[None]


def kernel(queries, res2, res3, res4, res5, W_q, b_q, Wv2, bv2, Wv3, bv3,
           Wv4, bv4, Wv5, bv5, level_embed, W_value, b_value, W_off, b_off,
           W_attn, b_attn, W_ao, b_ao, W_out, b_out):
    wv_cat = jnp.concatenate([Wv2, Wv3, Wv4, Wv5], axis=0)
    bias4 = jnp.stack([bv2, bv3, bv4, bv5]) + level_embed
    bias8 = jnp.concatenate([bias4, jnp.zeros((4, HID), F32)], axis=0)
    m_cat, c8 = _fold_call(wv_cat, bias8, W_value, b_value.reshape(1, HID))

    feats = (res2, res3, res4, res5)
    r_blocks = (2048, 1024, 1024, 256)
    tabs = []
    start = 0
    for l in range(NL):
        C, S = CDIMS[l], SIZES[l]
        m_l = lax.slice(m_cat, (start, 0), (start + C, HID))
        c_l = lax.slice(c8, (l, 0), (l + 1, HID))
        x = feats[l].reshape(B, C, S * S)
        v4 = _val_call(x, m_l, c_l, r_blocks[l], S)
        tabs.append(v4.reshape(B * NH * S * S, 2, DH))
        start += C

    wox = W_off[:, _PERM_OFF_X]
    box = b_off[_PERM_OFF_X].reshape(1, 128)
    woy = W_off[:, _PERM_OFF_Y]
    boy = b_off[_PERM_OFF_Y].reshape(1, 128)
    wat = W_attn[:, _PERM_ATTN]
    bat = b_attn[_PERM_ATTN].reshape(1, 128)
    idx_arr, w_arr = _prep_call(
        queries, W_q, b_q.reshape(1, HID), wox, box, woy, boy,
        wat, bat, jnp.asarray(_G_NP), jnp.asarray(_CENT_NP),
        jnp.asarray(_BOUND_NP), jnp.asarray(_SW_NP),
        jnp.asarray(_HWC_NP), jnp.asarray(_HCOL_NP))

    sampled = _sc_gather(tabs, idx_arr.reshape(2, B, NL, NQ * 32),
                         w_arr.reshape(4, B, NL, NQ * 32))

    out = _out_call(sampled.reshape(B * NQ, HID), W_ao, b_ao.reshape(1, HID),
                    W_out, b_out.reshape(1, 2560))
    return out.reshape(B, NQ, 2560)


# speedup vs baseline: 1.4580x; 1.0290x over previous
import functools

import numpy as np
import jax
import jax.numpy as jnp
from jax import lax
from jax.experimental import pallas as pl
from jax.experimental.pallas import tpu as pltpu
from jax.experimental.pallas import tpu_sc as plsc

F32 = jnp.float32
I32 = jnp.int32

B = 4
NQ = 256
NH = 8
NL = 4
NP = 4
HID = 256
DH = 32
SIZES = (128, 64, 32, 16)
CDIMS = (128, 256, 512, 1024)
NW = 32
QPW = NQ // NW


def _fold_body(wv_ref, bias_ref, wval_ref, bval_ref, m_ref, c_ref):
    wval = wval_ref[...]
    m_ref[...] = jnp.dot(wv_ref[...], wval, preferred_element_type=F32)
    c_ref[...] = jnp.dot(bias_ref[...], wval, preferred_element_type=F32) + bval_ref[...]


def _fold_call(wv_cat, bias8, w_value, b_value):
    return pl.pallas_call(
        _fold_body,
        out_shape=(
            jax.ShapeDtypeStruct((sum(CDIMS), HID), F32),
            jax.ShapeDtypeStruct((8, HID), F32),
        ),
    )(wv_cat, bias8, w_value, b_value)


def _val_body(s, x_ref, xt_ref, m_ref, c_ref, o_ref):
    acc = lax.dot_general(x_ref[0], m_ref[...], (((0,), (0,)), ((), ())),
                          preferred_element_type=F32) + c_ref[...]
    acct = lax.dot_general(xt_ref[0, :, :s], m_ref[...], (((0,), (0,)), ((), ())),
                           preferred_element_type=F32) + c_ref[...]
    R = acc.shape[0]
    nxt = jnp.concatenate([acc[s:], acct], axis=0)
    for hp in range(NH // 2):
        c0 = slice(hp * 2 * DH, hp * 2 * DH + DH)
        c1 = slice(hp * 2 * DH + DH, hp * 2 * DH + 2 * DH)
        o_ref[0, hp] = jnp.concatenate(
            [acc[:, c0], nxt[:, c0], acc[:, c1], nxt[:, c1]], axis=1)


def _val_call(x, m, c, R, S):
    _, C, HW = x.shape
    k = R // 128
    jmax = HW // 128 - 1
    return pl.pallas_call(
        functools.partial(_val_body, S),
        grid=(B, HW // R),
        in_specs=[
            pl.BlockSpec((1, C, R), lambda b, i: (b, 0, i)),
            pl.BlockSpec((1, C, 128), lambda b, i: (b, 0, jnp.minimum(i * k + k, jmax))),
            pl.BlockSpec((C, HID), lambda b, i: (0, 0)),
            pl.BlockSpec((1, HID), lambda b, i: (0, 0)),
        ],
        out_specs=pl.BlockSpec((1, NH // 2, R, 4 * DH), lambda b, i: (b, 0, i, 0)),
        out_shape=jax.ShapeDtypeStruct((B, NH // 2, HW, 4 * DH), F32),
    )(x, x, m, c)


def _prep_body(q_ref, wq_ref, bq_ref, wox_ref, box_ref, woy_ref, boy_ref,
               wat_ref, bat_ref, g_ref, cent_ref, bound_ref, sw_ref, hwc_ref,
               hcol_ref, i_ref, w_ref):
    b = pl.program_id(0)
    q = jnp.dot(q_ref[0], wq_ref[...], preferred_element_type=F32) + bq_ref[...]
    ox = jnp.dot(q, wox_ref[...], preferred_element_type=F32) + box_ref[...]
    oy = jnp.dot(q, woy_ref[...], preferred_element_type=F32) + boy_ref[...]
    logit = jnp.dot(q, wat_ref[...], preferred_element_type=F32) + bat_ref[...]
    m = jnp.max(logit, axis=1, keepdims=True)
    e = jnp.exp(logit - m)
    denom = jnp.dot(e, g_ref[...], preferred_element_type=F32)
    a = e / denom

    cent = cent_ref[...]
    bnd = bound_ref[...]
    ix = ox + cent
    iy = oy + cent
    x0 = jnp.floor(ix)
    y0 = jnp.floor(iy)
    fx = ix - x0
    fy = iy - y0
    one = jnp.float32(1.0)
    x1 = x0 + one
    y1 = y0 + one

    vx0 = ((x0 >= 0) & (x0 <= bnd)).astype(F32)
    vx1 = ((x1 >= 0) & (x1 <= bnd)).astype(F32)
    vy0 = ((y0 >= 0) & (y0 <= bnd)).astype(F32)
    vy1 = ((y1 >= 0) & (y1 <= bnd)).astype(F32)

    x0c = jnp.clip(x0, 0.0, bnd)
    x1c = jnp.clip(x1, 0.0, bnd)
    y0c = jnp.clip(y0, 0.0, bnd)
    y1c = jnp.clip(y1, 0.0, bnd)
    yb = jnp.clip(y0, 0.0, bnd - one)
    gA = ((one - fy) * vy0 * (y0c == yb).astype(F32)
          + fy * vy1 * (y1c == yb).astype(F32))
    gB = ((one - fy) * vy0 * (y0c == yb + one).astype(F32)
          + fy * vy1 * (y1c == yb + one).astype(F32))

    hcol = hcol_ref[...]
    hwc = hwc_ref[...]
    sw = sw_ref[...]
    base = ((b * (NH // 2) + jnp.right_shift(hcol, 1)) * (2 * hwc)
            + yb.astype(I32) * (2 * sw) + jnp.bitwise_and(hcol, 1))
    idxs = (base + x0c.astype(I32) * 2, base + x1c.astype(I32) * 2)
    wts = (a * (one - fx) * vx0 * gA,
           a * (one - fx) * vx0 * gB,
           a * fx * vx1 * gA,
           a * fx * vx1 * gB)

    for l in range(NL):
        sl = slice(l * 32, (l + 1) * 32)
        for c in range(2):
            i_ref[c, 0, l] = idxs[c][:, sl]
        for c in range(4):
            w_ref[c, 0, l] = wts[c][:, sl]


def _prep_call(q3, wq, bq, wox, box, woy, boy, wat, bat, g, cent, bound, sw,
               hwc, hcol):
    full = lambda shape: pl.BlockSpec(shape, lambda b: tuple(0 for _ in shape))
    ispec = pl.BlockSpec((2, 1, NL, NQ, 32), lambda b: (0, b, 0, 0, 0))
    wspec = pl.BlockSpec((4, 1, NL, NQ, 32), lambda b: (0, b, 0, 0, 0))
    return pl.pallas_call(
        _prep_body,
        grid=(B,),
        in_specs=[
            pl.BlockSpec((1, NQ, 2560), lambda b: (b, 0, 0)),
            full((2560, HID)), full((1, HID)),
            full((HID, 128)), full((1, 128)),
            full((HID, 128)), full((1, 128)),
            full((HID, 128)), full((1, 128)),
            full((128, 128)),
            full((1, 128)), full((1, 128)), full((1, 128)), full((1, 128)),
            full((1, 128)),
        ],
        out_specs=(ispec, wspec),
        out_shape=(jax.ShapeDtypeStruct((2, B, NL, NQ, 32), I32),
                   jax.ShapeDtypeStruct((4, B, NL, NQ, 32), F32)),
    )(q3, wq, bq, wox, box, woy, boy, wat, bat, g, cent, bound, sw, hwc, hcol)


_GDN = lax.GatherDimensionNumbers(offset_dims=(), collapsed_slice_dims=(0,),
                                  start_index_map=(0,))


def _lane_bcast(vec, j):
    idx = jnp.full((16, 1), j, dtype=I32)
    return lax.gather(vec, idx, _GDN, (1,),
                      mode=lax.GatherScatterMode.PROMISE_IN_BOUNDS)


_NSTEP = NL * B


def _sc_body(t2, t3, t4, t5, ihbm, whbm, out_ref,
             ivs, wvs, g, acc,
             si0, si1, si2, sw0, sw1, sw2,
             sg00, sg01, sg10, sg11):
    tabs = (t2, t3, t4, t5)
    si = (si0, si1, si2)
    sw = (sw0, sw1, sw2)
    sg = ((sg00, sg01), (sg10, sg11))

    wid = lax.axis_index("s") * 2 + lax.axis_index("c")
    q0m32 = wid * (QPW * 32)

    def stage(t):
        ss = t % 3
        b, l = t % 4, t // 4
        pltpu.async_copy(ihbm.at[:, b, l, pl.ds(q0m32, 256)], ivs.at[ss], si[ss])
        pltpu.async_copy(whbm.at[:, b, l, pl.ds(q0m32, 256)], wvs.at[ss], sw[ss])

    def fire(t):
        ss, gb, l = t % 3, t % 2, t // 4
        pltpu.make_async_copy(ihbm.at[:, 0, 0, pl.ds(0, 256)], ivs.at[ss],
                              si[ss]).wait()
        pltpu.make_async_copy(whbm.at[:, 0, 0, pl.ds(0, 256)], wvs.at[ss],
                              sw[ss]).wait()
        for c in range(2):
            for h4 in range(4):
                pltpu.async_copy(tabs[l].at[ivs.at[ss, c, pl.ds(h4 * 64, 64)]],
                                 g.at[gb, c, pl.ds(h4 * 64, 64)], sg[gb][c])

    def compute(t):
        ss, gb, l, b = t % 3, t % 2, t // 4, t % 4
        for c in range(2):
            pltpu.make_async_copy(tabs[0].at[pl.ds(0, 256)], g.at[gb, c],
                                  sg[gb][c]).wait()

        def kb(k, _):
            wvecs = [wvs[ss, c, pl.ds(k * 16, 16)] for c in range(4)]
            for rr in range(4):
                arow = b * 64 + k * 4 + rr
                lo = []
                hi = []
                for c in range(4):
                    cy, pr = c >> 1, c & 1
                    j0 = rr * 4
                    w0 = _lane_bcast(wvecs[c], j0)
                    cl = w0 * g[gb, cy, k * 16 + j0, pr, pl.ds(0, 16)]
                    ch = w0 * g[gb, cy, k * 16 + j0, pr, pl.ds(16, 16)]
                    for p in range(1, 4):
                        j = rr * 4 + p
                        wj = _lane_bcast(wvecs[c], j)
                        cl = cl + wj * g[gb, cy, k * 16 + j, pr, pl.ds(0, 16)]
                        ch = ch + wj * g[gb, cy, k * 16 + j, pr, pl.ds(16, 16)]
                    lo.append(cl)
                    hi.append(ch)
                al = (lo[0] + lo[1]) + (lo[2] + lo[3])
                ah = (hi[0] + hi[1]) + (hi[2] + hi[3])
                if l > 0:
                    al = al + acc[arow, pl.ds(0, 16)]
                    ah = ah + acc[arow, pl.ds(16, 16)]
                acc[arow, pl.ds(0, 16)] = al
                acc[arow, pl.ds(16, 16)] = ah
            return _

        lax.fori_loop(0, 16, kb, None)

    for t in range(_NSTEP + 2):
        if 1 <= t <= _NSTEP:
            fire(t - 1)
        if t < _NSTEP:
            stage(t)
        if t >= 2:
            compute(t - 2)

    for b in range(B):
        pltpu.sync_copy(acc.at[pl.ds(b * 64, 64)],
                        out_ref.at[pl.ds(b * 2048 + wid * 64, 64)])


def _sc_gather(tabs, ihbm, whbm):
    mesh = plsc.VectorSubcoreMesh(core_axis_name="c", subcore_axis_name="s")
    kern = pl.kernel(
        _sc_body,
        out_type=jax.ShapeDtypeStruct((B * NQ * NH, DH), F32),
        mesh=mesh,
        compiler_params=pltpu.CompilerParams(use_tc_tiling_on_sc=False),
        scratch_types=(
            [pltpu.VMEM((3, 2, 256), I32),
             pltpu.VMEM((3, 4, 256), F32),
             pltpu.VMEM((2, 2, 256, 2, DH), F32),
             pltpu.VMEM((B * 64, DH), F32)]
            + [pltpu.SemaphoreType.DMA for _ in range(10)]
        ),
    )
    return kern(*tabs, ihbm, whbm)


def _out_body(x_ref, wao_ref, bao_ref, wout_ref, bout_ref, o_ref):
    t = jnp.dot(x_ref[...], wao_ref[...], preferred_element_type=F32) + bao_ref[...]
    o_ref[...] = jnp.dot(t, wout_ref[...], preferred_element_type=F32) + bout_ref[...]


def _out_call(x, wao, bao, wout, bout):
    return pl.pallas_call(
        _out_body,
        out_shape=jax.ShapeDtypeStruct((B * NQ, 2560), F32),
    )(x, wao, bao, wout, bout)


_COLS = np.arange(128)
_L_OF = _COLS // 32
_H_OF = (_COLS % 32) // 4
_P_OF = _COLS % 4
_PERM_ATTN = _H_OF * 16 + _L_OF * 4 + _P_OF
_PERM_OFF_X = ((_H_OF * 4 + _L_OF) * 4 + _P_OF) * 2
_PERM_OFF_Y = _PERM_OFF_X + 1
_G_NP = (_H_OF[:, None] == _H_OF[None, :]).astype(np.float32)
_S_NP = np.array(SIZES, np.float32)[_L_OF]
_CENT_NP = (_S_NP / 2.0 - 0.5).astype(np.float32)[None]
_BOUND_NP = (_S_NP - 1.0).astype(np.float32)[None]
_SW_NP = _S_NP.astype(np.int32)[None]
_HWC_NP = (_S_NP * _S_NP).astype(np.int32)[None]
_HCOL_NP = _H_OF.astype(np.int32)[None]


def kernel(queries, res2, res3, res4, res5, W_q, b_q, Wv2, bv2, Wv3, bv3,
           Wv4, bv4, Wv5, bv5, level_embed, W_value, b_value, W_off, b_off,
           W_attn, b_attn, W_ao, b_ao, W_out, b_out):
    wv_cat = jnp.concatenate([Wv2, Wv3, Wv4, Wv5], axis=0)
    bias4 = jnp.stack([bv2, bv3, bv4, bv5]) + level_embed
    bias8 = jnp.concatenate([bias4, jnp.zeros((4, HID), F32)], axis=0)
    m_cat, c8 = _fold_call(wv_cat, bias8, W_value, b_value.reshape(1, HID))

    feats = (res2, res3, res4, res5)
    r_blocks = (4096, 2048, 1024, 256)
    tabs = []
    start = 0
    for l in range(NL):
        C, S = CDIMS[l], SIZES[l]
        m_l = lax.slice(m_cat, (start, 0), (start + C, HID))
        c_l = lax.slice(c8, (l, 0), (l + 1, HID))
        x = feats[l].reshape(B, C, S * S)
        v4 = _val_call(x, m_l, c_l, r_blocks[l], S)
        tabs.append(v4.reshape(B * NH * S * S, 2, DH))
        start += C

    wox = W_off[:, _PERM_OFF_X]
    box = b_off[_PERM_OFF_X].reshape(1, 128)
    woy = W_off[:, _PERM_OFF_Y]
    boy = b_off[_PERM_OFF_Y].reshape(1, 128)
    wat = W_attn[:, _PERM_ATTN]
    bat = b_attn[_PERM_ATTN].reshape(1, 128)
    idx_arr, w_arr = _prep_call(
        queries, W_q, b_q.reshape(1, HID), wox, box, woy, boy,
        wat, bat, jnp.asarray(_G_NP), jnp.asarray(_CENT_NP),
        jnp.asarray(_BOUND_NP), jnp.asarray(_SW_NP),
        jnp.asarray(_HWC_NP), jnp.asarray(_HCOL_NP))

    sampled = _sc_gather(tabs, idx_arr.reshape(2, B, NL, NQ * 32),
                         w_arr.reshape(4, B, NL, NQ * 32))

    out = _out_call(sampled.reshape(B * NQ, HID), W_ao, b_ao.reshape(1, HID),
                    W_out, b_out.reshape(1, 2560))
    return out.reshape(B, NQ, 2560)
